# baseline (speedup 1.0000x reference)
import functools

import jax
import jax.numpy as jnp
from jax.experimental import pallas as pl
from jax.experimental.pallas import tpu as pltpu

NUM_EXPERTS = 64
TOP_K = 8
HIDDEN = 4096
TOKENS = 32768

TOKEN_BLOCK = 1024


def _router_block_kernel(x_ref, w_ref, probs_ref, scores_ref, idx_ref):
    x = x_ref[...]
    w = w_ref[...]
    logits = jax.lax.dot_general(
        x, w,
        dimension_numbers=(((1,), (1,)), ((), ())),
        preferred_element_type=jnp.float32,
    )

    t = logits.shape[0]
    niota_f = -jax.lax.broadcasted_iota(
        jnp.int32, (t, NUM_EXPERTS), 1
    ).astype(jnp.float32)
    work = logits
    tops = []
    for _ in range(TOP_K):
        cur = jnp.max(work, axis=-1, keepdims=True)
        tops.append(cur)
        work = jnp.where(work == cur, jnp.float32(-jnp.inf), work)

    nidxs = [
        jnp.max(
            jnp.where(logits == cur, niota_f, -jnp.inf),
            axis=-1, keepdims=True,
        )
        for cur in tops
    ]
    top_logits = jnp.concatenate(tops, axis=-1)
    top_idx = (-jnp.concatenate(nidxs, axis=-1)).astype(jnp.int32)

    m = tops[0]
    e = jnp.exp(logits - m)
    denom = jnp.sum(e, axis=-1, keepdims=True)
    probs_ref[...] = e / denom

    e_top = jnp.exp(top_logits - m)
    scores_ref[...] = e_top / jnp.sum(e_top, axis=-1, keepdims=True)
    idx_ref[...] = top_idx


@jax.jit
def kernel(hidden_states, weight):
    n_tokens = hidden_states.shape[0]
    grid = (n_tokens // TOKEN_BLOCK,)
    probs, scores, idx = pl.pallas_call(
        _router_block_kernel,
        grid=grid,
        in_specs=[
            pl.BlockSpec((TOKEN_BLOCK, HIDDEN), lambda i: (i, 0)),
            pl.BlockSpec((NUM_EXPERTS, HIDDEN), lambda i: (0, 0)),
        ],
        out_specs=[
            pl.BlockSpec((TOKEN_BLOCK, NUM_EXPERTS), lambda i: (i, 0)),
            pl.BlockSpec((TOKEN_BLOCK, TOP_K), lambda i: (i, 0)),
            pl.BlockSpec((TOKEN_BLOCK, TOP_K), lambda i: (i, 0)),
        ],
        out_shape=[
            jax.ShapeDtypeStruct((n_tokens, NUM_EXPERTS), jnp.float32),
            jax.ShapeDtypeStruct((n_tokens, TOP_K), jnp.float32),
            jax.ShapeDtypeStruct((n_tokens, TOP_K), jnp.int32),
        ],
    )(hidden_states, weight)
    return (probs, scores, idx)


# speedup vs baseline: 1.0455x; 1.0455x over previous
import functools

import jax
import jax.numpy as jnp
from jax.experimental import pallas as pl
from jax.experimental.pallas import tpu as pltpu

NUM_EXPERTS = 64
TOP_K = 8
HIDDEN = 4096
TOKENS = 32768

TOKEN_BLOCK = 1024


def _router_block_kernel(x_ref, w_ref, probs_ref, scores_ref, idx_ref):
    x = x_ref[...]
    w = w_ref[...]
    logits = jax.lax.dot_general(
        x, w,
        dimension_numbers=(((1,), (1,)), ((), ())),
        preferred_element_type=jnp.float32,
    )

    t = logits.shape[0]
    lane_iota = jax.lax.broadcasted_iota(jnp.int32, (t, NUM_EXPERTS), 1)
    bits = jax.lax.bitcast_convert_type(logits, jnp.int32)
    keys = jax.lax.bitcast_convert_type(
        ((bits + 32) & ~63) | (63 - lane_iota), jnp.float32
    )
    work = keys
    tops = []
    for _ in range(TOP_K):
        cur = jnp.max(work, axis=-1, keepdims=True)
        tops.append(cur)
        work = jnp.where(work == cur, jnp.float32(-jnp.inf), work)

    top_keys = jax.lax.bitcast_convert_type(
        jnp.concatenate(tops, axis=-1), jnp.int32
    )
    top_idx = 63 - (top_keys & 63)
    top_logits = jax.lax.bitcast_convert_type(top_keys & ~63, jnp.float32)

    m = tops[0]
    e = jnp.exp(logits - m)
    denom = jnp.sum(e, axis=-1, keepdims=True)
    probs_ref[...] = e / denom

    e_top = jnp.exp(top_logits - m)
    scores_ref[...] = e_top / jnp.sum(e_top, axis=-1, keepdims=True)
    idx_ref[...] = top_idx


@jax.jit
def kernel(hidden_states, weight):
    n_tokens = hidden_states.shape[0]
    grid = (n_tokens // TOKEN_BLOCK,)
    probs, scores, idx = pl.pallas_call(
        _router_block_kernel,
        grid=grid,
        in_specs=[
            pl.BlockSpec((TOKEN_BLOCK, HIDDEN), lambda i: (i, 0)),
            pl.BlockSpec((NUM_EXPERTS, HIDDEN), lambda i: (0, 0)),
        ],
        out_specs=[
            pl.BlockSpec((TOKEN_BLOCK, NUM_EXPERTS), lambda i: (i, 0)),
            pl.BlockSpec((TOKEN_BLOCK, TOP_K), lambda i: (i, 0)),
            pl.BlockSpec((TOKEN_BLOCK, TOP_K), lambda i: (i, 0)),
        ],
        out_shape=[
            jax.ShapeDtypeStruct((n_tokens, NUM_EXPERTS), jnp.float32),
            jax.ShapeDtypeStruct((n_tokens, TOP_K), jnp.float32),
            jax.ShapeDtypeStruct((n_tokens, TOP_K), jnp.int32),
        ],
    )(hidden_states, weight)
    return (probs, scores, idx)
